# Initial kernel scaffold; baseline (speedup 1.0000x reference)
import jax
import jax.numpy as jnp
from jax.experimental import pallas as pl


def kernel(bias_params):
    raise NotImplementedError("write your pallas kernel here")


# speedup vs baseline: 9.6392x; 9.6392x over previous
import functools

import jax
import jax.numpy as jnp
from jax import lax
from jax.experimental import pallas as pl
from jax.experimental.pallas import tpu as pltpu
from jax.experimental.pallas import tpu_sc as plsc

_NUM_CLUSTER = 1024
_NUM_HEAD = 16
_KEY_LEN = 3 * _NUM_CLUSTER
_ROW_WORDS = _KEY_LEN * _NUM_HEAD
_TABLE_WORDS = (4 * _NUM_CLUSTER - 1) * _NUM_HEAD

_NUM_WORKERS = 32
_ROWS_PER_WORKER = _NUM_CLUSTER // _NUM_WORKERS


def _sc_broadcast(table_flat):
    mesh = plsc.VectorSubcoreMesh(core_axis_name="c", subcore_axis_name="s")

    @functools.partial(
        pl.kernel,
        mesh=mesh,
        out_type=jax.ShapeDtypeStruct((_NUM_CLUSTER * _ROW_WORDS,), jnp.float32),
        scratch_types=[
            pltpu.VMEM((_TABLE_WORDS,), jnp.float32),
            pltpu.SemaphoreType.DMA,
        ],
    )
    def k(table_hbm, out_hbm, table_v, sem):
        wid = lax.axis_index("s") * 2 + lax.axis_index("c")
        pltpu.sync_copy(table_hbm, table_v)
        base = wid * _ROWS_PER_WORKER
        copies = []
        for i in range(_ROWS_PER_WORKER):
            q = base + i
            src_off = (_NUM_CLUSTER - 1) * _NUM_HEAD - q * _NUM_HEAD
            copy = pltpu.make_async_copy(
                table_v.at[pl.ds(src_off, _ROW_WORDS)],
                out_hbm.at[pl.ds(q * _ROW_WORDS, _ROW_WORDS)],
                sem,
            )
            copy.start()
            copies.append(copy)
        for copy in copies:
            copy.wait()

    return k(table_flat)


def kernel(bias_params):
    out_flat = _sc_broadcast(bias_params.reshape(-1))
    return out_flat.reshape(_NUM_CLUSTER, _KEY_LEN, _NUM_HEAD)
